# baseline (speedup 1.0000x reference)
import dataclasses
import functools

import jax
import jax.numpy as jnp
from jax import lax
from jax.experimental import pallas as pl
from jax.experimental.pallas import tpu as pltpu
from jax.experimental.pallas import tpu_sc as plsc

N = 10000
E = 640000
IN = 26
LM = 512
HID = 128
OUT = 128

NPAD = 10240
TPAD = 10240
CT = 512
NCHUNK = TPAD // CT

NW = 32
ROWS_PER_TILE = NPAD // 16

EB = 128
EPAD0 = 32 * 157 * EB
NCH0 = 157
EMP = 32 * 160 * EB
NCHM = 160

_MESH = dict(core_axis_name="c", subcore_axis_name="s")


def _sc_params():
    cp = pltpu.CompilerParams()
    if "needs_layout_passes" in pltpu.CompilerParams.__dataclass_fields__:
        cp = dataclasses.replace(cp, needs_layout_passes=False)
    return cp


def _scan1_body(x_ref, wih0_ref, whh0_ref, b0_ref, out_ref, a0_ref, state_ref):
    i = pl.program_id(0)

    @pl.when(i == 0)
    def _():
        state_ref[...] = jnp.zeros((8, LM), jnp.float32)

    a0_ref[...] = (jnp.dot(x_ref[...], wih0_ref[...],
                           preferred_element_type=jnp.float32) + b0_ref[...])

    def step(t, carry):
        h1, c1 = carry
        g1 = a0_ref[pl.ds(t, 1), :] + jnp.dot(
            h1.astype(jnp.bfloat16), whh0_ref[...],
            preferred_element_type=jnp.float32)
        i1 = jax.nn.sigmoid(g1[:, 0 * LM:1 * LM])
        f1 = jax.nn.sigmoid(g1[:, 1 * LM:2 * LM])
        gg1 = jnp.tanh(g1[:, 2 * LM:3 * LM])
        o1 = jax.nn.sigmoid(g1[:, 3 * LM:4 * LM])
        c1 = f1 * c1 + i1 * gg1
        h1 = o1 * jnp.tanh(c1)
        out_ref[pl.ds(t, 1), :] = h1
        return h1, c1

    def step2(u, carry):
        return step(2 * u + 1, step(2 * u, carry))

    h1, c1 = lax.fori_loop(0, CT // 2, step2,
                           (state_ref[0:1, :], state_ref[1:2, :]))
    state_ref[0:1, :] = h1
    state_ref[1:2, :] = c1


def _scan1(x128, wih0t, whh0t, b0):
    const = lambda s: pl.BlockSpec(s, lambda i: tuple(0 for _ in s))
    return pl.pallas_call(
        _scan1_body,
        grid=(NCHUNK,),
        in_specs=[
            pl.BlockSpec((CT, 128), lambda i: (i, 0)),
            const((128, 4 * LM)),
            const((LM, 4 * LM)),
            const((1, 4 * LM)),
        ],
        out_specs=pl.BlockSpec((CT, LM), lambda i: (i, 0)),
        out_shape=jax.ShapeDtypeStruct((TPAD, LM), jnp.float32),
        scratch_shapes=[
            pltpu.VMEM((CT, 4 * LM), jnp.float32),
            pltpu.VMEM((8, LM), jnp.float32),
        ],
        compiler_params=pltpu.CompilerParams(
            dimension_semantics=("arbitrary",)),
    )(x128, wih0t, whh0t, b0)


def _scan2_body(h1_ref, x_ref, wih1_ref, whh1_ref, b1_ref,
                aa_ref, lm_ref, lmb_ref, w1t_ref, out_ref,
                a1_ref, h2seq_ref, state_ref):
    i = pl.program_id(0)

    @pl.when(i == 0)
    def _():
        state_ref[...] = jnp.zeros((8, LM), jnp.float32)

    a1_ref[...] = (jnp.dot(h1_ref[...].astype(jnp.bfloat16), wih1_ref[...],
                           preferred_element_type=jnp.float32) + b1_ref[...])

    def step(t, carry):
        h2, c2 = carry
        g2 = a1_ref[pl.ds(t, 1), :] + jnp.dot(
            h2.astype(jnp.bfloat16), whh1_ref[...],
            preferred_element_type=jnp.float32)
        i2 = jax.nn.sigmoid(g2[:, 0 * LM:1 * LM])
        f2 = jax.nn.sigmoid(g2[:, 1 * LM:2 * LM])
        gg2 = jnp.tanh(g2[:, 2 * LM:3 * LM])
        o2 = jax.nn.sigmoid(g2[:, 3 * LM:4 * LM])
        c2 = f2 * c2 + i2 * gg2
        h2 = o2 * jnp.tanh(c2)
        h2seq_ref[pl.ds(t, 1), :] = h2
        return h2, c2

    def step2(u, carry):
        return step(2 * u + 1, step(2 * u, carry))

    h2, c2 = lax.fori_loop(0, CT // 2, step2,
                           (state_ref[0:1, :], state_ref[1:2, :]))
    state_ref[0:1, :] = h2
    state_ref[1:2, :] = c2

    z = jnp.dot(x_ref[...], aa_ref[...], preferred_element_type=jnp.float32)
    z = z + jnp.dot(h2seq_ref[...], lm_ref[...],
                    preferred_element_type=jnp.float32) + lmb_ref[...]
    z = jnp.maximum(z, 0.0)
    out_ref[...] = jnp.dot(z, w1t_ref[...], preferred_element_type=jnp.float32)


def _scan2(h1seq, x128, wih1t, whh1t, b1v, aat, lmt, lmb, w1t):
    const = lambda s: pl.BlockSpec(s, lambda i: tuple(0 for _ in s))
    return pl.pallas_call(
        _scan2_body,
        grid=(NCHUNK,),
        in_specs=[
            pl.BlockSpec((CT, LM), lambda i: (i, 0)),
            pl.BlockSpec((CT, 128), lambda i: (i, 0)),
            const((LM, 4 * LM)),
            const((LM, 4 * LM)),
            const((1, 4 * LM)),
            const((128, LM)),
            const((LM, LM)),
            const((1, LM)),
            const((LM, HID)),
        ],
        out_specs=pl.BlockSpec((CT, HID), lambda i: (i, 0)),
        out_shape=jax.ShapeDtypeStruct((TPAD, HID), jnp.float32),
        scratch_shapes=[
            pltpu.VMEM((CT, 4 * LM), jnp.float32),
            pltpu.VMEM((CT, LM), jnp.float32),
            pltpu.VMEM((8, LM), jnp.float32),
        ],
        compiler_params=pltpu.CompilerParams(
            dimension_semantics=("arbitrary",)),
    )(h1seq, x128, wih1t, whh1t, b1v, aat, lmt, lmb, w1t)


def _deg_body(col_hbm, ew_hbm, zero_hbm, out_hbm, idx_v, ew_v, stage_v, acc_sh):
    cid = lax.axis_index("c")
    sid = lax.axis_index("s")
    wid = sid * 2 + cid
    pltpu.sync_copy(zero_hbm, acc_sh.at[pl.ds(sid * ROWS_PER_TILE,
                                              ROWS_PER_TILE)])
    plsc.subcore_barrier()

    base = wid * (NCH0 * EB)

    @pl.loop(0, NCH0)
    def _(j):
        off = base + j * EB
        pltpu.sync_copy(col_hbm.at[pl.ds(off, EB)], idx_v)
        pltpu.sync_copy(ew_hbm.at[pl.ds(off, EB)], ew_v)
        pltpu.sync_copy(ew_v, acc_sh.at[idx_v], add=True)

    plsc.subcore_barrier()
    pltpu.sync_copy(acc_sh.at[pl.ds(sid * ROWS_PER_TILE, ROWS_PER_TILE)],
                    stage_v)
    pltpu.sync_copy(stage_v, out_hbm.at[cid, pl.ds(sid * ROWS_PER_TILE,
                                                   ROWS_PER_TILE)])


def _deg_kernel(colp, ewp, zeros1):
    k = pl.kernel(
        _deg_body,
        out_type=jax.ShapeDtypeStruct((2, NPAD), jnp.float32),
        mesh=plsc.VectorSubcoreMesh(**_MESH),
        scratch_types=[
            pltpu.VMEM((EB,), jnp.int32),
            pltpu.VMEM((EB,), jnp.float32),
            pltpu.VMEM((ROWS_PER_TILE,), jnp.float32),
            pltpu.VMEM_SHARED((NPAD,), jnp.float32),
        ],
        compiler_params=_sc_params(),
    )
    return k(colp, ewp, zeros1)


def _dinv_body(degp_ref, dinv_ref, snorm_ref):
    deg = degp_ref[0:1, :] + degp_ref[1:2, :] + 1.0
    d = lax.rsqrt(deg)
    dinv_ref[...] = d
    snorm_ref[...] = d * d


def _dinv_kernel(degp):
    return pl.pallas_call(
        _dinv_body,
        out_shape=(jax.ShapeDtypeStruct((1, NPAD), jnp.float32),
                   jax.ShapeDtypeStruct((1, NPAD), jnp.float32)),
    )(degp)


def _norm_body(row_hbm, col_hbm, ew_hbm, dinv_hbm, out_hbm,
               dinv_v, row_v, col_v, ew_v, nrm_v):
    cid = lax.axis_index("c")
    sid = lax.axis_index("s")
    wid = sid * 2 + cid
    pltpu.sync_copy(dinv_hbm, dinv_v)
    base = wid * (NCH0 * EB)

    @pl.loop(0, NCH0)
    def _(j):
        off = base + j * EB
        pltpu.sync_copy(row_hbm.at[pl.ds(off, EB)], row_v)
        pltpu.sync_copy(col_hbm.at[pl.ds(off, EB)], col_v)
        pltpu.sync_copy(ew_hbm.at[pl.ds(off, EB)], ew_v)
        for g in range(EB // 16):
            sl = pl.ds(g * 16, 16)
            dr = plsc.load_gather(dinv_v, [row_v[sl]])
            dc = plsc.load_gather(dinv_v, [col_v[sl]])
            nrm_v[sl] = dr * ew_v[sl] * dc
        pltpu.sync_copy(nrm_v, out_hbm.at[pl.ds(off, EB)])


def _norm_kernel(rowp, colp, ewp, dinv):
    k = pl.kernel(
        _norm_body,
        out_type=jax.ShapeDtypeStruct((EPAD0,), jnp.float32),
        mesh=plsc.VectorSubcoreMesh(**_MESH),
        scratch_types=[
            pltpu.VMEM((NPAD,), jnp.float32),
            pltpu.VMEM((EB,), jnp.int32),
            pltpu.VMEM((EB,), jnp.int32),
            pltpu.VMEM((EB,), jnp.float32),
            pltpu.VMEM((EB,), jnp.float32),
        ],
        compiler_params=_sc_params(),
    )
    return k(rowp, colp, ewp, dinv)


def _msg_body(xw_hbm, row_hbm, col_hbm, nrm_hbm, zero_hbm, out_hbm,
              row_v, col_v, nrm_v, rows_v, acc_sh):
    cid = lax.axis_index("c")
    sid = lax.axis_index("s")
    wid = sid * 2 + cid
    pltpu.sync_copy(zero_hbm,
                    acc_sh.at[pl.ds(sid * ROWS_PER_TILE, ROWS_PER_TILE)])
    plsc.subcore_barrier()

    base = wid * (NCHM * EB)

    @pl.loop(0, NCHM)
    def _(j):
        off = base + j * EB
        pltpu.sync_copy(row_hbm.at[pl.ds(off, EB)], row_v)
        pltpu.sync_copy(col_hbm.at[pl.ds(off, EB)], col_v)
        pltpu.sync_copy(nrm_hbm.at[pl.ds(off, EB)], nrm_v)
        pltpu.sync_copy(xw_hbm.at[row_v], rows_v)

        @pl.loop(0, EB)
        def _(r):
            s = plsc.load_gather(nrm_v, [jnp.full((16,), r, jnp.int32)])
            for g in range(HID // 16):
                sl = pl.ds(g * 16, 16)
                rows_v[r, sl] = rows_v[r, sl] * s

        pltpu.sync_copy(rows_v, acc_sh.at[col_v], add=True)

    plsc.subcore_barrier()
    for k in range(ROWS_PER_TILE // EB):
        sl = pl.ds(sid * ROWS_PER_TILE + k * EB, EB)
        pltpu.sync_copy(acc_sh.at[sl], rows_v)
        pltpu.sync_copy(rows_v, out_hbm.at[cid, sl])


def _msg_kernel(xw, rowm, colm, nrmm, zeros2):
    k = pl.kernel(
        _msg_body,
        out_type=jax.ShapeDtypeStruct((2, NPAD, HID), jnp.float32),
        mesh=plsc.VectorSubcoreMesh(**_MESH),
        scratch_types=[
            pltpu.VMEM((EB,), jnp.int32),
            pltpu.VMEM((EB,), jnp.int32),
            pltpu.VMEM((EB,), jnp.float32),
            pltpu.VMEM((EB, HID), jnp.float32),
            pltpu.VMEM_SHARED((NPAD, HID), jnp.float32),
        ],
        compiler_params=_sc_params(),
    )
    return k(xw, rowm, colm, nrmm, zeros2)


def _mid_body(parts_ref, b_ref, wt_ref, out_ref):
    s = parts_ref[0] + parts_ref[1] + b_ref[...]
    s = jnp.maximum(s, 0.0)
    out_ref[...] = jnp.dot(s, wt_ref[...], preferred_element_type=jnp.float32)


def _mid_kernel(parts, b, wt):
    cs = 1024
    return pl.pallas_call(
        _mid_body,
        grid=(NPAD // cs,),
        in_specs=[
            pl.BlockSpec((2, cs, HID), lambda i: (0, i, 0)),
            pl.BlockSpec((1, HID), lambda i: (0, 0)),
            pl.BlockSpec((HID, HID), lambda i: (0, 0)),
        ],
        out_specs=pl.BlockSpec((cs, HID), lambda i: (i, 0)),
        out_shape=jax.ShapeDtypeStruct((NPAD, HID), jnp.float32),
    )(parts, b, wt)


def _final_body(parts_ref, b_ref, out_ref):
    out_ref[...] = parts_ref[0] + parts_ref[1] + b_ref[...]


def _final_kernel(parts, b):
    cs = 1024
    return pl.pallas_call(
        _final_body,
        grid=(NPAD // cs,),
        in_specs=[
            pl.BlockSpec((2, cs, HID), lambda i: (0, i, 0)),
            pl.BlockSpec((1, HID), lambda i: (0, 0)),
        ],
        out_specs=pl.BlockSpec((cs, HID), lambda i: (i, 0)),
        out_shape=jax.ShapeDtypeStruct((NPAD, HID), jnp.float32),
    )(parts, b)


def kernel(x, edge_index, edge_weight,
           w_ih0, w_hh0, b_ih0, b_hh0,
           w_ih1, w_hh1, b_ih1, b_hh1,
           aa_W, lm_W, lm_b,
           W1, b1, W2, b2, W3, b3):
    f32 = jnp.float32
    x128 = jnp.pad(x, ((0, TPAD - N), (0, 128 - IN)))
    wih0t = jnp.pad(w_ih0.T, ((0, 128 - IN), (0, 0)))
    whh0t = w_hh0.T.astype(jnp.bfloat16)
    wih1t = w_ih1.T.astype(jnp.bfloat16)
    whh1t = w_hh1.T.astype(jnp.bfloat16)
    b0 = (b_ih0 + b_hh0).reshape(1, 4 * LM)
    b1v = (b_ih1 + b_hh1).reshape(1, 4 * LM)
    aat = jnp.pad(aa_W.T, ((0, 128 - IN), (0, 0)))
    lmt = lm_W.T
    lmb = lm_b.reshape(1, LM)
    w1t = W1.T
    w2t = W2.T
    w3t = W3.T

    row = edge_index[0]
    col = edge_index[1]
    rowp = jnp.pad(row, (0, EPAD0 - E))
    colp = jnp.pad(col, (0, EPAD0 - E))
    ewp = jnp.pad(edge_weight, (0, EPAD0 - E))
    zeros1 = jnp.zeros((ROWS_PER_TILE,), f32)
    zeros2 = jnp.zeros((ROWS_PER_TILE, HID), f32)

    degp = _deg_kernel(colp, ewp, zeros1)
    dinv2d, snorm2d = _dinv_kernel(degp)
    dinv = dinv2d.reshape(NPAD)
    snorm = snorm2d.reshape(NPAD)

    nrm_e = _norm_kernel(rowp, colp, ewp, dinv)[:E]
    loop = jnp.arange(N, dtype=jnp.int32)
    rowm = jnp.pad(jnp.concatenate([row, loop]), (0, EMP - E - N))
    colm = jnp.pad(jnp.concatenate([col, loop]), (0, EMP - E - N))
    nrmm = jnp.pad(jnp.concatenate([nrm_e, snorm[:N]]), (0, EMP - E - N))

    h1seq = _scan1(x128, wih0t, whh0t, b0)
    xw1 = _scan2(h1seq, x128, wih1t, whh1t, b1v, aat, lmt, lmb, w1t)

    parts1 = _msg_kernel(xw1, rowm, colm, nrmm, zeros2)
    xw2 = _mid_kernel(parts1, b1.reshape(1, HID), w2t)
    parts2 = _msg_kernel(xw2, rowm, colm, nrmm, zeros2)
    xw3 = _mid_kernel(parts2, b2.reshape(1, HID), w3t)
    parts3 = _msg_kernel(xw3, rowm, colm, nrmm, zeros2)
    out = _final_kernel(parts3, b3.reshape(1, HID))
    return out[:N]


# speedup vs baseline: 1.3881x; 1.3881x over previous
import dataclasses
import functools

import jax
import jax.numpy as jnp
from jax import lax
from jax.experimental import pallas as pl
from jax.experimental.pallas import tpu as pltpu
from jax.experimental.pallas import tpu_sc as plsc

N = 10000
E = 640000
IN = 26
LM = 512
HID = 128
OUT = 128

NPAD = 10240
TPAD = 10240
CT = 512
NCHUNK = TPAD // CT

NW = 32
ROWS_PER_TILE = NPAD // 16

EB = 128
EPAD0 = 32 * 157 * EB
NCH0 = 157
EMP = 32 * 160 * EB
NCHM = 160
EMPX = EMP + 2 * EB

_MESH = dict(core_axis_name="c", subcore_axis_name="s")


def _sc_params():
    cp = pltpu.CompilerParams()
    if "needs_layout_passes" in pltpu.CompilerParams.__dataclass_fields__:
        cp = dataclasses.replace(cp, needs_layout_passes=False)
    return cp


def _scan1_body(x_ref, wih0_ref, whh0_ref, b0_ref, out_ref, a0_ref, state_ref):
    i = pl.program_id(0)

    @pl.when(i == 0)
    def _():
        state_ref[...] = jnp.zeros((8, LM), jnp.float32)

    a0_ref[...] = (jnp.dot(x_ref[...], wih0_ref[...],
                           preferred_element_type=jnp.float32) + b0_ref[...])

    def step(t, carry):
        h1, c1 = carry
        g1 = a0_ref[pl.ds(t, 1), :] + jnp.dot(
            h1.astype(jnp.bfloat16), whh0_ref[...],
            preferred_element_type=jnp.float32)
        i1 = jax.nn.sigmoid(g1[:, 0 * LM:1 * LM])
        f1 = jax.nn.sigmoid(g1[:, 1 * LM:2 * LM])
        gg1 = jnp.tanh(g1[:, 2 * LM:3 * LM])
        o1 = jax.nn.sigmoid(g1[:, 3 * LM:4 * LM])
        c1 = f1 * c1 + i1 * gg1
        h1 = o1 * jnp.tanh(c1)
        out_ref[pl.ds(t, 1), :] = h1
        return h1, c1

    def step2(u, carry):
        return step(2 * u + 1, step(2 * u, carry))

    h1, c1 = lax.fori_loop(0, CT // 2, step2,
                           (state_ref[0:1, :], state_ref[1:2, :]))
    state_ref[0:1, :] = h1
    state_ref[1:2, :] = c1


def _scan1(x128, wih0t, whh0t, b0):
    const = lambda s: pl.BlockSpec(s, lambda i: tuple(0 for _ in s))
    return pl.pallas_call(
        _scan1_body,
        grid=(NCHUNK,),
        in_specs=[
            pl.BlockSpec((CT, 128), lambda i: (i, 0)),
            const((128, 4 * LM)),
            const((LM, 4 * LM)),
            const((1, 4 * LM)),
        ],
        out_specs=pl.BlockSpec((CT, LM), lambda i: (i, 0)),
        out_shape=jax.ShapeDtypeStruct((TPAD, LM), jnp.float32),
        scratch_shapes=[
            pltpu.VMEM((CT, 4 * LM), jnp.float32),
            pltpu.VMEM((8, LM), jnp.float32),
        ],
        compiler_params=pltpu.CompilerParams(
            dimension_semantics=("arbitrary",)),
    )(x128, wih0t, whh0t, b0)


def _scan2_body(h1_ref, x_ref, wih1_ref, whh1_ref, b1_ref,
                aa_ref, lm_ref, lmb_ref, w1t_ref, out_ref,
                a1_ref, h2seq_ref, state_ref):
    i = pl.program_id(0)

    @pl.when(i == 0)
    def _():
        state_ref[...] = jnp.zeros((8, LM), jnp.float32)

    a1_ref[...] = (jnp.dot(h1_ref[...].astype(jnp.bfloat16), wih1_ref[...],
                           preferred_element_type=jnp.float32) + b1_ref[...])

    def step(t, carry):
        h2, c2 = carry
        g2 = a1_ref[pl.ds(t, 1), :] + jnp.dot(
            h2.astype(jnp.bfloat16), whh1_ref[...],
            preferred_element_type=jnp.float32)
        i2 = jax.nn.sigmoid(g2[:, 0 * LM:1 * LM])
        f2 = jax.nn.sigmoid(g2[:, 1 * LM:2 * LM])
        gg2 = jnp.tanh(g2[:, 2 * LM:3 * LM])
        o2 = jax.nn.sigmoid(g2[:, 3 * LM:4 * LM])
        c2 = f2 * c2 + i2 * gg2
        h2 = o2 * jnp.tanh(c2)
        h2seq_ref[pl.ds(t, 1), :] = h2
        return h2, c2

    def step2(u, carry):
        return step(2 * u + 1, step(2 * u, carry))

    h2, c2 = lax.fori_loop(0, CT // 2, step2,
                           (state_ref[0:1, :], state_ref[1:2, :]))
    state_ref[0:1, :] = h2
    state_ref[1:2, :] = c2

    z = jnp.dot(x_ref[...], aa_ref[...], preferred_element_type=jnp.float32)
    z = z + jnp.dot(h2seq_ref[...], lm_ref[...],
                    preferred_element_type=jnp.float32) + lmb_ref[...]
    z = jnp.maximum(z, 0.0)
    out_ref[...] = jnp.dot(z, w1t_ref[...], preferred_element_type=jnp.float32)


def _scan2(h1seq, x128, wih1t, whh1t, b1v, aat, lmt, lmb, w1t):
    const = lambda s: pl.BlockSpec(s, lambda i: tuple(0 for _ in s))
    return pl.pallas_call(
        _scan2_body,
        grid=(NCHUNK,),
        in_specs=[
            pl.BlockSpec((CT, LM), lambda i: (i, 0)),
            pl.BlockSpec((CT, 128), lambda i: (i, 0)),
            const((LM, 4 * LM)),
            const((LM, 4 * LM)),
            const((1, 4 * LM)),
            const((128, LM)),
            const((LM, LM)),
            const((1, LM)),
            const((LM, HID)),
        ],
        out_specs=pl.BlockSpec((CT, HID), lambda i: (i, 0)),
        out_shape=jax.ShapeDtypeStruct((TPAD, HID), jnp.float32),
        scratch_shapes=[
            pltpu.VMEM((CT, 4 * LM), jnp.float32),
            pltpu.VMEM((CT, LM), jnp.float32),
            pltpu.VMEM((8, LM), jnp.float32),
        ],
        compiler_params=pltpu.CompilerParams(
            dimension_semantics=("arbitrary",)),
    )(h1seq, x128, wih1t, whh1t, b1v, aat, lmt, lmb, w1t)


def _deg_body(col_hbm, ew_hbm, zero_hbm, out_hbm, idx_v, ew_v, stage_v, acc_sh):
    cid = lax.axis_index("c")
    sid = lax.axis_index("s")
    wid = sid * 2 + cid
    pltpu.sync_copy(zero_hbm, acc_sh.at[pl.ds(sid * ROWS_PER_TILE,
                                              ROWS_PER_TILE)])
    plsc.subcore_barrier()

    base = wid * (NCH0 * EB)

    @pl.loop(0, NCH0)
    def _(j):
        off = base + j * EB
        pltpu.sync_copy(col_hbm.at[pl.ds(off, EB)], idx_v)
        pltpu.sync_copy(ew_hbm.at[pl.ds(off, EB)], ew_v)
        pltpu.sync_copy(ew_v, acc_sh.at[idx_v], add=True)

    plsc.subcore_barrier()
    pltpu.sync_copy(acc_sh.at[pl.ds(sid * ROWS_PER_TILE, ROWS_PER_TILE)],
                    stage_v)
    pltpu.sync_copy(stage_v, out_hbm.at[cid, pl.ds(sid * ROWS_PER_TILE,
                                                   ROWS_PER_TILE)])


def _deg_kernel(colp, ewp, zeros1):
    k = pl.kernel(
        _deg_body,
        out_type=jax.ShapeDtypeStruct((2, NPAD), jnp.float32),
        mesh=plsc.VectorSubcoreMesh(**_MESH),
        scratch_types=[
            pltpu.VMEM((EB,), jnp.int32),
            pltpu.VMEM((EB,), jnp.float32),
            pltpu.VMEM((ROWS_PER_TILE,), jnp.float32),
            pltpu.VMEM_SHARED((NPAD,), jnp.float32),
        ],
        compiler_params=_sc_params(),
    )
    return k(colp, ewp, zeros1)


def _dinv_body(degp_ref, dinv_ref, snorm_ref):
    deg = degp_ref[0:1, :] + degp_ref[1:2, :] + 1.0
    d = lax.rsqrt(deg)
    dinv_ref[...] = d
    snorm_ref[...] = d * d


def _dinv_kernel(degp):
    return pl.pallas_call(
        _dinv_body,
        out_shape=(jax.ShapeDtypeStruct((1, NPAD), jnp.float32),
                   jax.ShapeDtypeStruct((1, NPAD), jnp.float32)),
    )(degp)


def _norm_body(row_hbm, col_hbm, ew_hbm, dinv_hbm, out_hbm,
               dinv_v, row_v, col_v, ew_v, nrm_v):
    cid = lax.axis_index("c")
    sid = lax.axis_index("s")
    wid = sid * 2 + cid
    pltpu.sync_copy(dinv_hbm, dinv_v)
    base = wid * (NCH0 * EB)

    @pl.loop(0, NCH0)
    def _(j):
        off = base + j * EB
        pltpu.sync_copy(row_hbm.at[pl.ds(off, EB)], row_v)
        pltpu.sync_copy(col_hbm.at[pl.ds(off, EB)], col_v)
        pltpu.sync_copy(ew_hbm.at[pl.ds(off, EB)], ew_v)
        for g in range(EB // 16):
            sl = pl.ds(g * 16, 16)
            dr = plsc.load_gather(dinv_v, [row_v[sl]])
            dc = plsc.load_gather(dinv_v, [col_v[sl]])
            nrm_v[sl] = dr * ew_v[sl] * dc
        pltpu.sync_copy(nrm_v, out_hbm.at[pl.ds(off, EB)])


def _norm_kernel(rowp, colp, ewp, dinv):
    k = pl.kernel(
        _norm_body,
        out_type=jax.ShapeDtypeStruct((EPAD0,), jnp.float32),
        mesh=plsc.VectorSubcoreMesh(**_MESH),
        scratch_types=[
            pltpu.VMEM((NPAD,), jnp.float32),
            pltpu.VMEM((EB,), jnp.int32),
            pltpu.VMEM((EB,), jnp.int32),
            pltpu.VMEM((EB,), jnp.float32),
            pltpu.VMEM((EB,), jnp.float32),
        ],
        compiler_params=_sc_params(),
    )
    return k(rowp, colp, ewp, dinv)


def _msg_body(xw_hbm, row_hbm, col_hbm, nrm_hbm, zero_hbm, out_hbm,
              row_v0, col_v0, nrm_v0, rows_v0,
              row_v1, col_v1, nrm_v1, rows_v1,
              sa0, sa1, sg0, sg1, acc_sh):
    cid = lax.axis_index("c")
    sid = lax.axis_index("s")
    wid = sid * 2 + cid
    pltpu.sync_copy(zero_hbm,
                    acc_sh.at[pl.ds(sid * ROWS_PER_TILE, ROWS_PER_TILE)])
    plsc.subcore_barrier()

    base = wid * (NCHM * EB)
    bufs = ((row_v0, col_v0, nrm_v0, rows_v0, sa0, sg0),
            (row_v1, col_v1, nrm_v1, rows_v1, sa1, sg1))

    def start_copies(j, b):
        off = base + j * EB
        row_v, col_v, nrm_v, _, sa, _ = bufs[b]
        pltpu.async_copy(row_hbm.at[pl.ds(off, EB)], row_v, sa)
        pltpu.async_copy(col_hbm.at[pl.ds(off, EB)], col_v, sa)
        pltpu.async_copy(nrm_hbm.at[pl.ds(off, EB)], nrm_v, sa)

    def wait_copies(b):
        row_v, col_v, nrm_v, _, sa, _ = bufs[b]
        pltpu.make_async_copy(row_hbm.at[pl.ds(0, EB)], row_v, sa).wait()
        pltpu.make_async_copy(col_hbm.at[pl.ds(0, EB)], col_v, sa).wait()
        pltpu.make_async_copy(nrm_hbm.at[pl.ds(0, EB)], nrm_v, sa).wait()

    def start_gather(b):
        row_v, _, _, rows_v, _, sg = bufs[b]
        pltpu.async_copy(xw_hbm.at[row_v], rows_v, sg)

    def wait_gather(b):
        row_v, _, _, rows_v, _, sg = bufs[b]
        pltpu.make_async_copy(xw_hbm.at[row_v], rows_v, sg).wait()

    def process(b):
        _, col_v, nrm_v, rows_v, _, _ = bufs[b]

        @pl.loop(0, EB)
        def _(r):
            s = plsc.load_gather(nrm_v, [jnp.full((16,), r, jnp.int32)])
            for g in range(HID // 16):
                sl = pl.ds(g * 16, 16)
                rows_v[r, sl] = rows_v[r, sl] * s

        pltpu.sync_copy(rows_v, acc_sh.at[col_v], add=True)

    start_copies(0, 0)
    wait_copies(0)
    start_gather(0)
    start_copies(1, 1)

    @pl.loop(0, NCHM // 2)
    def _(u):
        j = 2 * u
        for b in range(2):
            wait_copies(1 - b)
            start_gather(1 - b)
            wait_gather(b)
            process(b)
            start_copies(j + b + 2, b)

    wait_copies(1)
    wait_gather(0)

    plsc.subcore_barrier()
    for k in range(ROWS_PER_TILE // EB):
        sl = pl.ds(sid * ROWS_PER_TILE + k * EB, EB)
        pltpu.sync_copy(acc_sh.at[sl], rows_v0)
        pltpu.sync_copy(rows_v0, out_hbm.at[cid, sl])


def _msg_kernel(xw, rowm, colm, nrmm, zeros2):
    k = pl.kernel(
        _msg_body,
        out_type=jax.ShapeDtypeStruct((2, NPAD, HID), jnp.float32),
        mesh=plsc.VectorSubcoreMesh(**_MESH),
        scratch_types=[
            pltpu.VMEM((EB,), jnp.int32),
            pltpu.VMEM((EB,), jnp.int32),
            pltpu.VMEM((EB,), jnp.float32),
            pltpu.VMEM((EB, HID), jnp.float32),
            pltpu.VMEM((EB,), jnp.int32),
            pltpu.VMEM((EB,), jnp.int32),
            pltpu.VMEM((EB,), jnp.float32),
            pltpu.VMEM((EB, HID), jnp.float32),
            pltpu.SemaphoreType.DMA,
            pltpu.SemaphoreType.DMA,
            pltpu.SemaphoreType.DMA,
            pltpu.SemaphoreType.DMA,
            pltpu.VMEM_SHARED((NPAD, HID), jnp.float32),
        ],
        compiler_params=_sc_params(),
    )
    return k(xw, rowm, colm, nrmm, zeros2)


def _mid_body(parts_ref, b_ref, wt_ref, out_ref):
    s = parts_ref[0] + parts_ref[1] + b_ref[...]
    s = jnp.maximum(s, 0.0)
    out_ref[...] = jnp.dot(s, wt_ref[...], preferred_element_type=jnp.float32)


def _mid_kernel(parts, b, wt):
    cs = 1024
    return pl.pallas_call(
        _mid_body,
        grid=(NPAD // cs,),
        in_specs=[
            pl.BlockSpec((2, cs, HID), lambda i: (0, i, 0)),
            pl.BlockSpec((1, HID), lambda i: (0, 0)),
            pl.BlockSpec((HID, HID), lambda i: (0, 0)),
        ],
        out_specs=pl.BlockSpec((cs, HID), lambda i: (i, 0)),
        out_shape=jax.ShapeDtypeStruct((NPAD, HID), jnp.float32),
    )(parts, b, wt)


def _final_body(parts_ref, b_ref, out_ref):
    out_ref[...] = parts_ref[0] + parts_ref[1] + b_ref[...]


def _final_kernel(parts, b):
    cs = 1024
    return pl.pallas_call(
        _final_body,
        grid=(NPAD // cs,),
        in_specs=[
            pl.BlockSpec((2, cs, HID), lambda i: (0, i, 0)),
            pl.BlockSpec((1, HID), lambda i: (0, 0)),
        ],
        out_specs=pl.BlockSpec((cs, HID), lambda i: (i, 0)),
        out_shape=jax.ShapeDtypeStruct((NPAD, HID), jnp.float32),
    )(parts, b)


def kernel(x, edge_index, edge_weight,
           w_ih0, w_hh0, b_ih0, b_hh0,
           w_ih1, w_hh1, b_ih1, b_hh1,
           aa_W, lm_W, lm_b,
           W1, b1, W2, b2, W3, b3):
    f32 = jnp.float32
    x128 = jnp.pad(x, ((0, TPAD - N), (0, 128 - IN)))
    wih0t = jnp.pad(w_ih0.T, ((0, 128 - IN), (0, 0)))
    whh0t = w_hh0.T.astype(jnp.bfloat16)
    wih1t = w_ih1.T.astype(jnp.bfloat16)
    whh1t = w_hh1.T.astype(jnp.bfloat16)
    b0 = (b_ih0 + b_hh0).reshape(1, 4 * LM)
    b1v = (b_ih1 + b_hh1).reshape(1, 4 * LM)
    aat = jnp.pad(aa_W.T, ((0, 128 - IN), (0, 0)))
    lmt = lm_W.T
    lmb = lm_b.reshape(1, LM)
    w1t = W1.T
    w2t = W2.T
    w3t = W3.T

    row = edge_index[0]
    col = edge_index[1]
    rowp = jnp.pad(row, (0, EPAD0 - E))
    colp = jnp.pad(col, (0, EPAD0 - E))
    ewp = jnp.pad(edge_weight, (0, EPAD0 - E))
    zeros1 = jnp.zeros((ROWS_PER_TILE,), f32)
    zeros2 = jnp.zeros((ROWS_PER_TILE, HID), f32)

    degp = _deg_kernel(colp, ewp, zeros1)
    dinv2d, snorm2d = _dinv_kernel(degp)
    dinv = dinv2d.reshape(NPAD)
    snorm = snorm2d.reshape(NPAD)

    nrm_e = _norm_kernel(rowp, colp, ewp, dinv)[:E]
    loop = jnp.arange(N, dtype=jnp.int32)
    rowm = jnp.pad(jnp.concatenate([row, loop]), (0, EMPX - E - N))
    colm = jnp.pad(jnp.concatenate([col, loop]), (0, EMPX - E - N))
    nrmm = jnp.pad(jnp.concatenate([nrm_e, snorm[:N]]), (0, EMPX - E - N))

    h1seq = _scan1(x128, wih0t, whh0t, b0)
    xw1 = _scan2(h1seq, x128, wih1t, whh1t, b1v, aat, lmt, lmb, w1t)

    parts1 = _msg_kernel(xw1, rowm, colm, nrmm, zeros2)
    xw2 = _mid_kernel(parts1, b1.reshape(1, HID), w2t)
    parts2 = _msg_kernel(xw2, rowm, colm, nrmm, zeros2)
    xw3 = _mid_kernel(parts2, b2.reshape(1, HID), w3t)
    parts3 = _msg_kernel(xw3, rowm, colm, nrmm, zeros2)
    out = _final_kernel(parts3, b3.reshape(1, HID))
    return out[:N]
